# baseline (speedup 1.0000x reference)
import functools

import jax
import jax.numpy as jnp
from jax import lax
from jax.experimental import pallas as pl
from jax.experimental.pallas import tpu as pltpu
from jax.experimental.pallas import tpu_sc as plsc


def _p2w_body(p2_ref, w0a_ref, out_ref):
    out_ref[0] = jnp.dot(p2_ref[0], w0a_ref[...],
                         preferred_element_type=jnp.float32)


def _select_body(x1_ref, x2_ref, idx_ref, wgt_ref, *, M, b0):
    b = pl.program_id(0) + b0
    x1 = x1_ref[0]
    x2 = x2_ref[0]
    s = ((x1[:, 0:1] - x2[0:1, :]) ** 2
         + (x1[:, 1:2] - x2[1:2, :]) ** 2
         + (x1[:, 2:3] - x2[2:3, :]) ** 2)
    iota = lax.broadcasted_iota(jnp.int32, s.shape, 1)
    key = (s.view(jnp.int32) & jnp.int32(~(M - 1))) | iota
    ibig = jnp.int32(0x7FFFFFFF)
    m1 = jnp.min(key, axis=1, keepdims=True)
    ka = jnp.where(key <= m1, ibig, key)
    m2 = jnp.min(ka, axis=1, keepdims=True)
    kb = jnp.where(ka <= m2, ibig, ka)
    m3 = jnp.min(kb, axis=1, keepdims=True)
    lo = jnp.int32(M - 1)
    i1, i2, i3 = m1 & lo, m2 & lo, m3 & lo
    d1 = (m1 & ~lo).view(jnp.float32)
    d2 = (m2 & ~lo).view(jnp.float32)
    d3 = (m3 & ~lo).view(jnp.float32)
    w1 = 1.0 / jnp.maximum(d1, 1e-10)
    w2 = 1.0 / jnp.maximum(d2, 1e-10)
    w3 = 1.0 / jnp.maximum(d3, 1e-10)
    inorm = 1.0 / (w1 + w2 + w3)
    base = b * M
    idx_ref[0] = jnp.concatenate([i1 + base, i2 + base, i3 + base], axis=1)
    bn = w1.shape[0]
    wgt_ref[0] = jnp.concatenate([
        jnp.broadcast_to(w1 * inorm, (bn, 16)),
        jnp.broadcast_to(w2 * inorm, (bn, 16)),
        jnp.broadcast_to(w3 * inorm, (bn, 16)),
    ], axis=1)


def _mlp_body(hp_ref, p1_ref, w0b_ref, w1_ref, out_ref):
    h = jnp.maximum(
        hp_ref[0] + jnp.dot(p1_ref[0], w0b_ref[...],
                            preferred_element_type=jnp.float32), 0.0)
    out_ref[0] = jnp.maximum(
        jnp.dot(h, w1_ref[...], preferred_element_type=jnp.float32), 0.0)


def _sc_interp(p2w_flat, idx_flat, wgt48, F):
    info = plsc.get_sparse_core_info()
    NW = info.num_cores * info.num_subcores
    npts = wgt48.shape[0]
    CP = 32
    per_w = npts // NW
    n_chunks = per_w // CP
    NFC = F // 16

    mesh = plsc.VectorSubcoreMesh(core_axis_name="c", subcore_axis_name="s")

    @functools.partial(
        pl.kernel, mesh=mesh,
        out_type=jax.ShapeDtypeStruct((npts, F), jnp.float32),
        scratch_types=[
            pltpu.VMEM((3 * CP,), jnp.int32),
            pltpu.VMEM((3 * CP, F), jnp.float32),
            pltpu.VMEM((CP, 48), jnp.float32),
            pltpu.VMEM((CP, F), jnp.float32),
            pltpu.SemaphoreType.DMA,
        ],
    )
    def k(p2w_hbm, idx_hbm, wgt_hbm, out_hbm,
          idx_v, rows_v, wgt_v, out_v, sem):
        wid = lax.axis_index("s") * info.num_cores + lax.axis_index("c")
        base = wid * per_w

        def chunk(c, carry):
            pbase = base + c * CP
            pltpu.sync_copy(idx_hbm.at[pl.ds(3 * pbase, 3 * CP)], idx_v)
            pltpu.sync_copy(wgt_hbm.at[pl.ds(pbase, CP)], wgt_v)
            pltpu.async_copy(p2w_hbm.at[idx_v], rows_v, sem).wait()

            @plsc.parallel_loop(0, CP, step=1, unroll=2)
            def point(p):
                wa = wgt_v[p, pl.ds(0, 16)]
                wb = wgt_v[p, pl.ds(16, 16)]
                wc = wgt_v[p, pl.ds(32, 16)]
                for fc in range(NFC):
                    fs = pl.ds(fc * 16, 16)
                    acc = (wa * rows_v[3 * p, fs]
                           + wb * rows_v[3 * p + 1, fs]
                           + wc * rows_v[3 * p + 2, fs])
                    out_v[p, fs] = acc
            pltpu.sync_copy(out_v, out_hbm.at[pl.ds(pbase, CP)])
            return carry

        lax.fori_loop(0, n_chunks, chunk, 0)

    return k(p2w_flat, idx_flat, wgt48)


@jax.jit
def kernel(xyz1, xyz2, points1, points2, W0, W1):
    B, N, _ = xyz1.shape
    M = xyz2.shape[1]
    C1 = points1.shape[2]
    C2 = points2.shape[2]
    F0 = W0.shape[1]
    F1 = W1.shape[1]
    W0a = W0[:C2]
    W0b = W0[C2:]
    xyz2t = jnp.transpose(xyz2, (0, 2, 1))

    p2w = pl.pallas_call(
        _p2w_body,
        grid=(B,),
        in_specs=[
            pl.BlockSpec((1, M, C2), lambda b: (b, 0, 0)),
            pl.BlockSpec((C2, F0), lambda b: (0, 0)),
        ],
        out_specs=pl.BlockSpec((1, M, F0), lambda b: (b, 0, 0)),
        out_shape=jax.ShapeDtypeStruct((B, M, F0), jnp.float32),
    )(points2, W0a)

    BN = 2048
    SL = 2
    Bh = B // SL
    p2w_flat = p2w.reshape(B * M, F0)
    outs = []
    for h in range(SL):
        sl = slice(h * Bh, (h + 1) * Bh)
        idx_h, wgt_h = pl.pallas_call(
            functools.partial(_select_body, M=M, b0=h * Bh),
            grid=(Bh, N // BN),
            in_specs=[
                pl.BlockSpec((1, BN, 3), lambda b, n: (b, n, 0)),
                pl.BlockSpec((1, 3, M), lambda b, n: (b, 0, 0)),
            ],
            out_specs=[
                pl.BlockSpec((1, BN, 3), lambda b, n: (b, n, 0)),
                pl.BlockSpec((1, BN, 48), lambda b, n: (b, n, 0)),
            ],
            out_shape=[
                jax.ShapeDtypeStruct((Bh, N, 3), jnp.int32),
                jax.ShapeDtypeStruct((Bh, N, 48), jnp.float32),
            ],
        )(xyz1[sl], xyz2t[sl])

        hp = _sc_interp(p2w_flat, idx_h.reshape(Bh * N * 3),
                        wgt_h.reshape(Bh * N, 48), F0)
        hp3 = hp.reshape(Bh, N, F0)

        outs.append(pl.pallas_call(
            _mlp_body,
            grid=(Bh, N // BN),
            in_specs=[
                pl.BlockSpec((1, BN, F0), lambda b, n: (b, n, 0)),
                pl.BlockSpec((1, BN, C1), lambda b, n: (b, n, 0)),
                pl.BlockSpec((C1, F0), lambda b, n: (0, 0)),
                pl.BlockSpec((F0, F1), lambda b, n: (0, 0)),
            ],
            out_specs=pl.BlockSpec((1, BN, F1), lambda b, n: (b, n, 0)),
            out_shape=jax.ShapeDtypeStruct((Bh, N, F1), jnp.float32),
        )(hp3, points1[sl], W0b, W1))
    return jnp.concatenate(outs, axis=0)


# speedup vs baseline: 1.1508x; 1.1508x over previous
import functools

import jax
import jax.numpy as jnp
from jax import lax
from jax.experimental import pallas as pl
from jax.experimental.pallas import tpu as pltpu
from jax.experimental.pallas import tpu_sc as plsc


def _p2w_body(p2_ref, w0a_ref, out_ref):
    out_ref[0] = jnp.dot(p2_ref[0], w0a_ref[...],
                         preferred_element_type=jnp.float32)


def _select_body(x1_ref, x2_ref, idx_ref, wgt_ref, *, M, b0):
    b = pl.program_id(0) + b0
    x1 = x1_ref[0]
    x2 = x2_ref[0]
    s = ((x1[:, 0:1] - x2[0:1, :]) ** 2
         + (x1[:, 1:2] - x2[1:2, :]) ** 2
         + (x1[:, 2:3] - x2[2:3, :]) ** 2)
    iota = lax.broadcasted_iota(jnp.int32, s.shape, 1)
    key = (s.view(jnp.int32) & jnp.int32(~(M - 1))) | iota
    ibig = jnp.int32(0x7FFFFFFF)
    m1 = jnp.min(key, axis=1, keepdims=True)
    ka = jnp.where(key <= m1, ibig, key)
    m2 = jnp.min(ka, axis=1, keepdims=True)
    kb = jnp.where(ka <= m2, ibig, ka)
    m3 = jnp.min(kb, axis=1, keepdims=True)
    lo = jnp.int32(M - 1)
    i1, i2, i3 = m1 & lo, m2 & lo, m3 & lo
    d1 = (m1 & ~lo).view(jnp.float32)
    d2 = (m2 & ~lo).view(jnp.float32)
    d3 = (m3 & ~lo).view(jnp.float32)
    w1 = 1.0 / jnp.maximum(d1, 1e-10)
    w2 = 1.0 / jnp.maximum(d2, 1e-10)
    w3 = 1.0 / jnp.maximum(d3, 1e-10)
    inorm = 1.0 / (w1 + w2 + w3)
    base = b * M
    idx_ref[0] = jnp.concatenate([i1 + base, i2 + base, i3 + base], axis=1)
    bn = w1.shape[0]
    wgt_ref[0] = jnp.concatenate([
        jnp.broadcast_to(w1 * inorm, (bn, 16)),
        jnp.broadcast_to(w2 * inorm, (bn, 16)),
        jnp.broadcast_to(w3 * inorm, (bn, 16)),
    ], axis=1)


def _mlp_body(hp_ref, p1_ref, w0b_ref, w1_ref, out_ref):
    h = jnp.maximum(
        hp_ref[0] + jnp.dot(p1_ref[0], w0b_ref[...],
                            preferred_element_type=jnp.float32), 0.0)
    out_ref[0] = jnp.maximum(
        jnp.dot(h, w1_ref[...], preferred_element_type=jnp.float32), 0.0)


def _sc_interp(p2w_flat, idx_flat, wgt48, F):
    info = plsc.get_sparse_core_info()
    NW = info.num_cores * info.num_subcores
    npts = wgt48.shape[0]
    CP = 32
    per_w = npts // NW
    n_chunks = per_w // CP
    NFC = F // 16

    mesh = plsc.VectorSubcoreMesh(core_axis_name="c", subcore_axis_name="s")

    @functools.partial(
        pl.kernel, mesh=mesh,
        out_type=jax.ShapeDtypeStruct((npts, F), jnp.float32),
        scratch_types=[
            pltpu.VMEM((3 * CP,), jnp.int32),
            pltpu.VMEM((3 * CP,), jnp.int32),
            pltpu.VMEM((3 * CP, F), jnp.float32),
            pltpu.VMEM((3 * CP, F), jnp.float32),
            pltpu.VMEM((CP, 48), jnp.float32),
            pltpu.VMEM((CP, 48), jnp.float32),
            pltpu.VMEM((CP, F), jnp.float32),
            pltpu.SemaphoreType.DMA,
            pltpu.SemaphoreType.DMA,
        ],
    )
    def k(p2w_hbm, idx_hbm, wgt_hbm, out_hbm,
          idx_v0, idx_v1, rows_v0, rows_v1, wgt_v0, wgt_v1, out_v,
          sem0, sem1):
        wid = lax.axis_index("s") * info.num_cores + lax.axis_index("c")
        base = wid * per_w
        idx_b = (idx_v0, idx_v1)
        rows_b = (rows_v0, rows_v1)
        wgt_b = (wgt_v0, wgt_v1)
        sem_b = (sem0, sem1)

        pltpu.sync_copy(idx_hbm.at[pl.ds(3 * base, 3 * CP)], idx_v0)
        pltpu.sync_copy(wgt_hbm.at[pl.ds(base, CP)], wgt_v0)
        pltpu.async_copy(p2w_hbm.at[idx_v0], rows_v0, sem0)

        def pair(i, carry):
            for par in range(2):
                c = 2 * i + par
                pbase = base + c * CP
                npar = 1 - par
                pltpu.make_async_copy(
                    p2w_hbm.at[idx_b[par]], rows_b[par], sem_b[par]).wait()

                @pl.when(c < n_chunks - 1)
                def _prefetch():
                    nbase = pbase + CP
                    pltpu.sync_copy(
                        idx_hbm.at[pl.ds(3 * nbase, 3 * CP)], idx_b[npar])
                    pltpu.sync_copy(
                        wgt_hbm.at[pl.ds(nbase, CP)], wgt_b[npar])
                    pltpu.async_copy(
                        p2w_hbm.at[idx_b[npar]], rows_b[npar], sem_b[npar])

                rows_v = rows_b[par]
                wgt_v = wgt_b[par]

                @plsc.parallel_loop(0, CP, step=1, unroll=2)
                def point(p):
                    wa = wgt_v[p, pl.ds(0, 16)]
                    wb = wgt_v[p, pl.ds(16, 16)]
                    wc = wgt_v[p, pl.ds(32, 16)]
                    for fc in range(NFC):
                        fs = pl.ds(fc * 16, 16)
                        acc = (wa * rows_v[3 * p, fs]
                               + wb * rows_v[3 * p + 1, fs]
                               + wc * rows_v[3 * p + 2, fs])
                        out_v[p, fs] = acc
                pltpu.sync_copy(out_v, out_hbm.at[pl.ds(pbase, CP)])
            return carry

        lax.fori_loop(0, n_chunks // 2, pair, 0)

    return k(p2w_flat, idx_flat, wgt48)


@jax.jit
def kernel(xyz1, xyz2, points1, points2, W0, W1):
    B, N, _ = xyz1.shape
    M = xyz2.shape[1]
    C1 = points1.shape[2]
    C2 = points2.shape[2]
    F0 = W0.shape[1]
    F1 = W1.shape[1]
    W0a = W0[:C2]
    W0b = W0[C2:]
    xyz2t = jnp.transpose(xyz2, (0, 2, 1))

    p2w = pl.pallas_call(
        _p2w_body,
        grid=(B,),
        in_specs=[
            pl.BlockSpec((1, M, C2), lambda b: (b, 0, 0)),
            pl.BlockSpec((C2, F0), lambda b: (0, 0)),
        ],
        out_specs=pl.BlockSpec((1, M, F0), lambda b: (b, 0, 0)),
        out_shape=jax.ShapeDtypeStruct((B, M, F0), jnp.float32),
    )(points2, W0a)

    BN = 2048
    SL = 4
    Bh = B // SL
    p2w_flat = p2w.reshape(B * M, F0)
    outs = []
    for h in range(SL):
        sl = slice(h * Bh, (h + 1) * Bh)
        idx_h, wgt_h = pl.pallas_call(
            functools.partial(_select_body, M=M, b0=h * Bh),
            grid=(Bh, N // BN),
            in_specs=[
                pl.BlockSpec((1, BN, 3), lambda b, n: (b, n, 0)),
                pl.BlockSpec((1, 3, M), lambda b, n: (b, 0, 0)),
            ],
            out_specs=[
                pl.BlockSpec((1, BN, 3), lambda b, n: (b, n, 0)),
                pl.BlockSpec((1, BN, 48), lambda b, n: (b, n, 0)),
            ],
            out_shape=[
                jax.ShapeDtypeStruct((Bh, N, 3), jnp.int32),
                jax.ShapeDtypeStruct((Bh, N, 48), jnp.float32),
            ],
        )(xyz1[sl], xyz2t[sl])

        hp = _sc_interp(p2w_flat, idx_h.reshape(Bh * N * 3),
                        wgt_h.reshape(Bh * N, 48), F0)
        hp3 = hp.reshape(Bh, N, F0)

        outs.append(pl.pallas_call(
            _mlp_body,
            grid=(Bh, N // BN),
            in_specs=[
                pl.BlockSpec((1, BN, F0), lambda b, n: (b, n, 0)),
                pl.BlockSpec((1, BN, C1), lambda b, n: (b, n, 0)),
                pl.BlockSpec((C1, F0), lambda b, n: (0, 0)),
                pl.BlockSpec((F0, F1), lambda b, n: (0, 0)),
            ],
            out_specs=pl.BlockSpec((1, BN, F1), lambda b, n: (b, n, 0)),
            out_shape=jax.ShapeDtypeStruct((Bh, N, F1), jnp.float32),
        )(hp3, points1[sl], W0b, W1))
    return jnp.concatenate(outs, axis=0)
